# baseline (speedup 1.0000x reference)
import functools

import jax
import jax.numpy as jnp
from jax.experimental import pallas as pl
from jax.experimental.pallas import tpu as pltpu

_EPS = 1e-5


def _fold(layer):
    W, b = layer["W"], layer["b"]
    if "gamma" in layer:
        s = layer["gamma"] * jax.lax.rsqrt(layer["var"] + _EPS)
        W = W * s[None, :]
        b = (b - layer["mean"]) * s + layer["beta"]
    return W, b


def _fps_kernel(xyz_ref, out_ref, *, npoint):
    x0 = xyz_ref[:, 0, :]
    x1 = xyz_ref[:, 1, :]
    x2 = xyz_ref[:, 2, :]
    Bsz, Np = x0.shape
    iota = jax.lax.broadcasted_iota(jnp.int32, (Bsz, Np), 1)

    def body(i, state):
        distance, farthest = state
        onehot = (iota == farthest).astype(jnp.float32)
        c0 = jnp.sum(onehot * x0, axis=1, keepdims=True)
        c1 = jnp.sum(onehot * x1, axis=1, keepdims=True)
        c2 = jnp.sum(onehot * x2, axis=1, keepdims=True)
        out_ref[pl.ds(i, 1)] = jnp.concatenate([c0, c1, c2], axis=1)[None]
        dist = (x0 - c0) ** 2 + (x1 - c1) ** 2 + (x2 - c2) ** 2
        distance = jnp.minimum(distance, dist)
        farthest = jnp.argmax(distance, axis=1).astype(jnp.int32)[:, None]
        return distance, farthest

    init = (jnp.full((Bsz, Np), 1e10, jnp.float32),
            jnp.zeros((Bsz, 1), jnp.int32))
    jax.lax.fori_loop(0, npoint, body, init)


def _fps(xyz_cn, npoint):
    Bsz, _, Np = xyz_cn.shape
    half = Bsz // 2
    out = pl.pallas_call(
        functools.partial(_fps_kernel, npoint=npoint),
        grid=(2,),
        in_specs=[pl.BlockSpec((half, 3, Np), lambda g: (g, 0, 0))],
        out_specs=pl.BlockSpec((npoint, half, 3), lambda g: (0, g, 0)),
        out_shape=jax.ShapeDtypeStruct((npoint, Bsz, 3), jnp.float32),
        compiler_params=pltpu.CompilerParams(
            dimension_semantics=("parallel",)),
    )(xyz_cn)
    return jnp.transpose(out, (1, 0, 2))


def _sa_kernel(xyz_ref, pts_ref, new_ref, *wb_refs, out_ref, radius, nsample,
               has_pts):
    xyz = xyz_ref[0]
    new = new_ref[0]
    Np = xyz.shape[0]
    CH = new.shape[0]

    sqr = (jnp.sum(new * new, axis=1, keepdims=True)
           + jnp.sum(xyz * xyz, axis=1)[None, :]
           - 2.0 * jax.lax.dot_general(
               new, xyz, (((1,), (1,)), ((), ())),
               preferred_element_type=jnp.float32))
    maskf = (sqr <= radius * radius).astype(jnp.float32)

    r_i = jax.lax.broadcasted_iota(jnp.int32, (Np, Np), 0)
    c_i = jax.lax.broadcasted_iota(jnp.int32, (Np, Np), 1)
    tri = (r_i < c_i).astype(jnp.float32)
    rank = jax.lax.dot_general(maskf, tri, (((1,), (0,)), ((), ())),
                               preferred_element_type=jnp.float32)
    rank = rank.astype(jnp.int32)
    count = rank[:, -1:] + maskf[:, -1:].astype(jnp.int32)

    jiota = jax.lax.broadcasted_iota(jnp.int32, (CH, nsample, Np), 1)
    sel = jnp.where((rank[:, None, :] == jiota) & (maskf[:, None, :] > 0.0),
                    1.0, 0.0)
    sel2 = sel.reshape(CH * nsample, Np)

    gx = jax.lax.dot_general(sel2, xyz, (((1,), (0,)), ((), ())),
                             preferred_element_type=jnp.float32)
    gx = gx.reshape(CH, nsample, 3)
    slot = jax.lax.broadcasted_iota(jnp.int32, (CH, nsample, 1), 1)
    valid = slot < count[:, :, None]
    gx = jnp.where(valid, gx, gx[:, 0:1, :])
    gx = gx - new[:, None, :]

    if has_pts:
        pts = pts_ref[0]
        gp = jax.lax.dot_general(sel2, pts, (((1,), (0,)), ((), ())),
                                 preferred_element_type=jnp.float32)
        gp = gp.reshape(CH, nsample, pts.shape[1])
        gp = jnp.where(valid, gp, gp[:, 0:1, :])
        h = jnp.concatenate([gx, gp], axis=2)
    else:
        h = gx
    h = h.reshape(CH * nsample, h.shape[2])

    for k in range(0, len(wb_refs), 2):
        W = wb_refs[k][...]
        b = wb_refs[k + 1][...]
        h = jax.lax.dot_general(h, W, (((1,), (0,)), ((), ())),
                                preferred_element_type=jnp.float32)
        h = jnp.maximum(h + b, 0.0)

    h = h.reshape(CH, nsample, h.shape[1])
    out_ref[0] = jnp.max(h, axis=1)


def _sa(xyz, pts, new_xyz, wbs, radius, nsample, chunk):
    Bsz, Np, _ = xyz.shape
    S = new_xyz.shape[1]
    Cout = wbs[-2].shape[1]
    nchunks = S // chunk
    has_pts = pts is not None
    Cpts = pts.shape[2] if has_pts else 0

    in_specs = [pl.BlockSpec((1, Np, 3), lambda b, c: (b, 0, 0))]
    ins = [xyz]
    if has_pts:
        in_specs.append(pl.BlockSpec((1, Np, Cpts), lambda b, c: (b, 0, 0)))
        ins.append(pts)
    in_specs.append(pl.BlockSpec((1, chunk, 3), lambda b, c: (b, c, 0)))
    ins.append(new_xyz)
    for wb in wbs:
        in_specs.append(pl.BlockSpec(wb.shape, lambda b, c: (0, 0)))
        ins.append(wb)

    def kern(*refs):
        if has_pts:
            xyz_ref, pts_ref, new_ref, *wb_refs = refs[:-1]
        else:
            xyz_ref, new_ref, *wb_refs = refs[:-1]
            pts_ref = None
        _sa_kernel(xyz_ref, pts_ref, new_ref, *wb_refs, out_ref=refs[-1],
                   radius=radius, nsample=nsample, has_pts=has_pts)

    return pl.pallas_call(
        kern,
        grid=(Bsz, nchunks),
        in_specs=in_specs,
        out_specs=pl.BlockSpec((1, chunk, Cout), lambda b, c: (b, c, 0)),
        out_shape=jax.ShapeDtypeStruct((Bsz, S, Cout), jnp.float32),
        compiler_params=pltpu.CompilerParams(
            dimension_semantics=("parallel", "parallel")),
    )(*ins)


def _head_kernel(xyz_ref, pts_ref, *wb_refs_and_out):
    *wb_refs, out_ref = wb_refs_and_out
    Bsz, S, Cp = pts_ref.shape
    h = jnp.concatenate([xyz_ref[...].reshape(Bsz * S, 3),
                         pts_ref[...].reshape(Bsz * S, Cp)], axis=1)
    for k in range(0, 6, 2):
        W = wb_refs[k][...]
        b = wb_refs[k + 1][...]
        h = jax.lax.dot_general(h, W, (((1,), (0,)), ((), ())),
                                preferred_element_type=jnp.float32)
        h = jnp.maximum(h + b, 0.0)
    h = jnp.max(h.reshape(Bsz, S, h.shape[1]), axis=1)
    for k in range(6, len(wb_refs), 2):
        W = wb_refs[k][...]
        b = wb_refs[k + 1][...]
        h = jax.lax.dot_general(h, W, (((1,), (0,)), ((), ())),
                                preferred_element_type=jnp.float32)
        h = h + b
        if k < len(wb_refs) - 2:
            h = jnp.maximum(h, 0.0)
    m = jnp.max(h, axis=1, keepdims=True)
    sh = h - m
    out_ref[...] = sh - jnp.log(jnp.sum(jnp.exp(sh), axis=1, keepdims=True))


def _head(xyz, pts, wbs, num_class):
    Bsz, S, Cp = pts.shape
    half = Bsz // 2
    in_specs = [pl.BlockSpec((half, S, 3), lambda g: (g, 0, 0)),
                pl.BlockSpec((half, S, Cp), lambda g: (g, 0, 0))]
    for wb in wbs:
        in_specs.append(pl.BlockSpec(wb.shape, lambda g: (0, 0)))
    return pl.pallas_call(
        _head_kernel,
        grid=(2,),
        in_specs=in_specs,
        out_specs=pl.BlockSpec((half, num_class), lambda g: (g, 0)),
        out_shape=jax.ShapeDtypeStruct((Bsz, num_class), jnp.float32),
        compiler_params=pltpu.CompilerParams(
            dimension_semantics=("parallel",)),
    )(xyz, pts, *wbs)


def kernel(xyz, params):
    x = jnp.transpose(xyz, (0, 2, 1))

    sa1_wbs = [a for l in params["sa1"] for a in _fold(l)]
    sa2_wbs = [a for l in params["sa2"] for a in _fold(l)]
    sa3_wbs = [a for l in params["sa3"] for a in _fold(l)]
    fc_wbs = [a for n in ("fc1", "fc2", "fc3") for a in _fold(params[n])]
    head_wbs = [w if w.ndim == 2 else w[None, :] for w in sa3_wbs + fc_wbs]
    sa1_wbs = [w if w.ndim == 2 else w[None, :] for w in sa1_wbs]
    sa2_wbs = [w if w.ndim == 2 else w[None, :] for w in sa2_wbs]

    l1_xyz = _fps(xyz, 512)
    l1_pts = _sa(x, None, l1_xyz, sa1_wbs, radius=0.2, nsample=32, chunk=64)
    l2_xyz = _fps(jnp.transpose(l1_xyz, (0, 2, 1)), 128)
    l2_pts = _sa(l1_xyz, l1_pts, l2_xyz, sa2_wbs, radius=0.4, nsample=64,
                 chunk=64)
    return _head(l2_xyz, l2_pts, head_wbs, 40)


# speedup vs baseline: 1.1244x; 1.1244x over previous
import functools

import jax
import jax.numpy as jnp
from jax import lax
from jax.experimental import pallas as pl
from jax.experimental.pallas import tpu as pltpu
from jax.experimental.pallas import tpu_sc as plsc

_EPS = 1e-5


def _fold(layer):
    W, b = layer["W"], layer["b"]
    if "gamma" in layer:
        s = layer["gamma"] * jax.lax.rsqrt(layer["var"] + _EPS)
        W = W * s[None, :]
        b = (b - layer["mean"]) * s + layer["beta"]
    return W, b


def _fps_kernel(xyz_ref, out_ref, *, npoint):
    x0 = xyz_ref[:, 0, :]
    x1 = xyz_ref[:, 1, :]
    x2 = xyz_ref[:, 2, :]
    Bsz, Np = x0.shape
    iota = jax.lax.broadcasted_iota(jnp.int32, (Bsz, Np), 1)

    def body(i, state):
        distance, farthest = state
        onehot = (iota == farthest).astype(jnp.float32)
        c0 = jnp.sum(onehot * x0, axis=1, keepdims=True)
        c1 = jnp.sum(onehot * x1, axis=1, keepdims=True)
        c2 = jnp.sum(onehot * x2, axis=1, keepdims=True)
        out_ref[pl.ds(i, 1)] = jnp.concatenate([c0, c1, c2], axis=1)[None]
        dist = (x0 - c0) ** 2 + (x1 - c1) ** 2 + (x2 - c2) ** 2
        distance = jnp.minimum(distance, dist)
        farthest = jnp.argmax(distance, axis=1).astype(jnp.int32)[:, None]
        return distance, farthest

    init = (jnp.full((Bsz, Np), 1e10, jnp.float32),
            jnp.zeros((Bsz, 1), jnp.int32))
    jax.lax.fori_loop(0, npoint, body, init)


def _fps(xyz_cn, npoint):
    Bsz = xyz_cn.shape[0]
    out = pl.pallas_call(
        functools.partial(_fps_kernel, npoint=npoint),
        out_shape=jax.ShapeDtypeStruct((npoint, Bsz, 3), jnp.float32),
    )(xyz_cn)
    return jnp.transpose(out, (1, 0, 2))


def _fps_sc(xyz_cn, npoint):
    Bsz, _, Np = xyz_cn.shape
    nsl = Np // 16
    mesh = plsc.VectorSubcoreMesh(core_axis_name="c", subcore_axis_name="s")

    @functools.partial(
        pl.kernel,
        mesh=mesh,
        out_type=jax.ShapeDtypeStruct((Bsz, npoint * 16), jnp.float32),
        scratch_types=[
            pltpu.VMEM((3 * Np,), jnp.float32),
            pltpu.VMEM((Np,), jnp.float32),
            pltpu.VMEM((npoint * 16,), jnp.float32),
        ],
    )
    def k(xyz_hbm, out_hbm, xyz_v, dist_v, res_v):
        wid = lax.axis_index("s") * 2 + lax.axis_index("c")
        pltpu.sync_copy(xyz_hbm.at[wid], xyz_v)
        lane = lax.iota(jnp.int32, 16)
        def shuf(v, idx):
            return v.at[idx].get(mode="promise_in_bounds")

        def bfly(v, op):
            for kk in (1, 2, 4, 8):
                v = op(v, shuf(v, lane ^ kk))
            return v

        def init_body(s, _):
            dist_v[pl.ds(s * 16, 16)] = jnp.full((16,), 1e10, jnp.float32)
            return 0
        lax.fori_loop(0, nsl, init_body, 0)

        z16 = jnp.zeros((16,), jnp.int32)
        c0_0 = shuf(xyz_v[pl.ds(0, 16)], z16)
        c1_0 = shuf(xyz_v[pl.ds(Np, 16)], z16)
        c2_0 = shuf(xyz_v[pl.ds(2 * Np, 16)], z16)

        def step(i, carry):
            c0, c1, c2 = carry
            row = jnp.where(lane == 0, c0, jnp.where(lane == 1, c1, c2))
            res_v[pl.ds(i * 16, 16)] = row

            def upd(s, ucarry):
                runmax, runidx, r0, r1, r2 = ucarry
                x0 = xyz_v[pl.ds(s * 16, 16)]
                x1 = xyz_v[pl.ds(Np + s * 16, 16)]
                x2 = xyz_v[pl.ds(2 * Np + s * 16, 16)]
                d0 = x0 - c0
                d1 = x1 - c1
                d2 = x2 - c2
                d = d0 * d0 + d1 * d1 + d2 * d2
                d = jnp.minimum(dist_v[pl.ds(s * 16, 16)], d)
                dist_v[pl.ds(s * 16, 16)] = d
                better = d > runmax
                runmax = jnp.where(better, d, runmax)
                runidx = jnp.where(better, s * 16 + lane, runidx)
                r0 = jnp.where(better, x0, r0)
                r1 = jnp.where(better, x1, r1)
                r2 = jnp.where(better, x2, r2)
                return runmax, runidx, r0, r1, r2

            zf = jnp.zeros((16,), jnp.float32)
            runmax, runidx, r0, r1, r2 = lax.fori_loop(
                0, nsl, upd,
                (jnp.full((16,), -1.0, jnp.float32), z16, zf, zf, zf))
            m = bfly(runmax, jnp.maximum)
            cand = jnp.where(runmax == m, runidx, Np)
            widx = bfly(cand, jnp.minimum)
            lwin = bfly(jnp.where(runidx == widx, lane, 16), jnp.minimum)
            return shuf(r0, lwin), shuf(r1, lwin), shuf(r2, lwin)

        lax.fori_loop(0, npoint, step, (c0_0, c1_0, c2_0))
        pltpu.sync_copy(res_v, out_hbm.at[wid])

    out = k(xyz_cn.reshape(Bsz, 3 * Np))
    return out.reshape(Bsz, npoint, 16)[:, :, :3]


def _sa_kernel(xyz_ref, pts_ref, new_ref, *wb_refs, out_ref, radius, nsample,
               has_pts):
    xyz = xyz_ref[0]
    new = new_ref[0]
    Np = xyz.shape[0]
    CH = new.shape[0]

    sqr = (jnp.sum(new * new, axis=1, keepdims=True)
           + jnp.sum(xyz * xyz, axis=1)[None, :]
           - 2.0 * jax.lax.dot_general(
               new, xyz, (((1,), (1,)), ((), ())),
               preferred_element_type=jnp.float32))
    maskf = (sqr <= radius * radius).astype(jnp.float32)

    G = Np // 128
    r_i = jax.lax.broadcasted_iota(jnp.int32, (128, 128), 0)
    c_i = jax.lax.broadcasted_iota(jnp.int32, (128, 128), 1)
    tri = (r_i < c_i).astype(jnp.float32)
    maskc = maskf.reshape(CH * G, 128)
    rloc = jax.lax.dot_general(maskc, tri, (((1,), (0,)), ((), ())),
                               preferred_element_type=jnp.float32)
    tot = (rloc[:, -1:] + maskc[:, -1:]).reshape(CH, G)
    g_r = jax.lax.broadcasted_iota(jnp.int32, (G, G), 0)
    g_c = jax.lax.broadcasted_iota(jnp.int32, (G, G), 1)
    trig = (g_r < g_c).astype(jnp.float32)
    pre = jax.lax.dot_general(tot, trig, (((1,), (0,)), ((), ())),
                              preferred_element_type=jnp.float32)
    rank = (rloc.reshape(CH, G, 128)
            + pre[:, :, None]).reshape(CH, Np).astype(jnp.int32)
    count = rank[:, -1:] + maskf[:, -1:].astype(jnp.int32)

    rankm = jnp.where(maskf > 0.0, rank, -1)
    jiota = jax.lax.broadcasted_iota(jnp.int32, (CH, nsample, Np), 1)
    sel = jnp.where(rankm[:, None, :] == jiota, 1.0, 0.0)
    sel2 = sel.reshape(CH * nsample, Np)

    gx = jax.lax.dot_general(sel2, xyz, (((1,), (0,)), ((), ())),
                             preferred_element_type=jnp.float32)
    gx = gx.reshape(CH, nsample, 3)
    slot = jax.lax.broadcasted_iota(jnp.int32, (CH, nsample, 1), 1)
    valid = slot < count[:, :, None]
    gx = jnp.where(valid, gx, gx[:, 0:1, :])
    gx = gx - new[:, None, :]

    if has_pts:
        pts = pts_ref[0]
        gp = jax.lax.dot_general(sel2, pts, (((1,), (0,)), ((), ())),
                                 preferred_element_type=jnp.float32)
        gp = gp.reshape(CH, nsample, pts.shape[1])
        gp = jnp.where(valid, gp, gp[:, 0:1, :])
        h = jnp.concatenate([gx, gp], axis=2)
    else:
        h = gx
    h = h.reshape(CH * nsample, h.shape[2])

    for k in range(0, len(wb_refs), 2):
        W = wb_refs[k][...]
        b = wb_refs[k + 1][...]
        h = jax.lax.dot_general(h, W, (((1,), (0,)), ((), ())),
                                preferred_element_type=jnp.float32)
        h = jnp.maximum(h + b, 0.0)

    h = h.reshape(CH, nsample, h.shape[1])
    out_ref[0] = jnp.max(h, axis=1)


def _sa(xyz, pts, new_xyz, wbs, radius, nsample, chunk):
    Bsz, Np, _ = xyz.shape
    S = new_xyz.shape[1]
    Cout = wbs[-2].shape[1]
    nchunks = S // chunk
    has_pts = pts is not None
    Cpts = pts.shape[2] if has_pts else 0

    in_specs = [pl.BlockSpec((1, Np, 3), lambda b, c: (b, 0, 0))]
    ins = [xyz]
    if has_pts:
        in_specs.append(pl.BlockSpec((1, Np, Cpts), lambda b, c: (b, 0, 0)))
        ins.append(pts)
    in_specs.append(pl.BlockSpec((1, chunk, 3), lambda b, c: (b, c, 0)))
    ins.append(new_xyz)
    for wb in wbs:
        in_specs.append(pl.BlockSpec(wb.shape, lambda b, c: (0, 0)))
        ins.append(wb)

    def kern(*refs):
        if has_pts:
            xyz_ref, pts_ref, new_ref, *wb_refs = refs[:-1]
        else:
            xyz_ref, new_ref, *wb_refs = refs[:-1]
            pts_ref = None
        _sa_kernel(xyz_ref, pts_ref, new_ref, *wb_refs, out_ref=refs[-1],
                   radius=radius, nsample=nsample, has_pts=has_pts)

    return pl.pallas_call(
        kern,
        grid=(Bsz, nchunks),
        in_specs=in_specs,
        out_specs=pl.BlockSpec((1, chunk, Cout), lambda b, c: (b, c, 0)),
        out_shape=jax.ShapeDtypeStruct((Bsz, S, Cout), jnp.float32),
    )(*ins)


def _head_kernel(xyz_ref, pts_ref, *wb_refs_and_out):
    *wb_refs, out_ref = wb_refs_and_out
    Bsz, S, Cp = pts_ref.shape
    h = jnp.concatenate([xyz_ref[...].reshape(Bsz * S, 3),
                         pts_ref[...].reshape(Bsz * S, Cp)], axis=1)
    for k in range(0, 6, 2):
        W = wb_refs[k][...]
        b = wb_refs[k + 1][...]
        h = jax.lax.dot_general(h, W, (((1,), (0,)), ((), ())),
                                preferred_element_type=jnp.float32)
        h = jnp.maximum(h + b, 0.0)
    h = jnp.max(h.reshape(Bsz, S, h.shape[1]), axis=1)
    for k in range(6, len(wb_refs), 2):
        W = wb_refs[k][...]
        b = wb_refs[k + 1][...]
        h = jax.lax.dot_general(h, W, (((1,), (0,)), ((), ())),
                                preferred_element_type=jnp.float32)
        h = h + b
        if k < len(wb_refs) - 2:
            h = jnp.maximum(h, 0.0)
    m = jnp.max(h, axis=1, keepdims=True)
    sh = h - m
    out_ref[...] = sh - jnp.log(jnp.sum(jnp.exp(sh), axis=1, keepdims=True))


def _head(xyz, pts, wbs, num_class):
    Bsz = xyz.shape[0]
    return pl.pallas_call(
        _head_kernel,
        out_shape=jax.ShapeDtypeStruct((Bsz, num_class), jnp.float32),
    )(xyz, pts, *wbs)


def kernel(xyz, params):
    x = jnp.transpose(xyz, (0, 2, 1))

    sa1_wbs = [a for l in params["sa1"] for a in _fold(l)]
    sa2_wbs = [a for l in params["sa2"] for a in _fold(l)]
    sa3_wbs = [a for l in params["sa3"] for a in _fold(l)]
    fc_wbs = [a for n in ("fc1", "fc2", "fc3") for a in _fold(params[n])]
    head_wbs = [w if w.ndim == 2 else w[None, :] for w in sa3_wbs + fc_wbs]
    sa1_wbs = [w if w.ndim == 2 else w[None, :] for w in sa1_wbs]
    sa2_wbs = [w if w.ndim == 2 else w[None, :] for w in sa2_wbs]

    l1_xyz = _fps(xyz, 512)
    l1_pts = _sa(x, None, l1_xyz, sa1_wbs, radius=0.2, nsample=32, chunk=64)
    l2_xyz = _fps_sc(jnp.transpose(l1_xyz, (0, 2, 1)), 128)
    l2_pts = _sa(l1_xyz, l1_pts, l2_xyz, sa2_wbs, radius=0.4, nsample=64,
                 chunk=64)
    return _head(l2_xyz, l2_pts, head_wbs, 40)
